# baseline (speedup 1.0000x reference)
import functools

import jax
import jax.numpy as jnp
from jax import lax
from jax.experimental import pallas as pl
from jax.experimental.pallas import tpu as pltpu
from jax.experimental.pallas import tpu_sc as plsc

_EMBED_DIM = 32
_NUM_CORES = 2
_NUM_SUBCORES = 16
_NUM_WORKERS = _NUM_CORES * _NUM_SUBCORES


def _make_gather(total_rows: int, chunk: int):
    assert total_rows % (_NUM_WORKERS * chunk) == 0
    rows_per_worker = total_rows // _NUM_WORKERS
    n_chunks = rows_per_worker // chunk
    mesh = plsc.VectorSubcoreMesh(core_axis_name="c", subcore_axis_name="s")

    @functools.partial(
        pl.kernel,
        mesh=mesh,
        compiler_params=pltpu.CompilerParams(use_tc_tiling_on_sc=False),
        out_type=jax.ShapeDtypeStruct((total_rows, _EMBED_DIM), jnp.float32),
        scratch_types=[
            pltpu.VMEM((chunk,), jnp.int32),
            pltpu.VMEM((chunk, _EMBED_DIM), jnp.float32),
            pltpu.SemaphoreType.DMA,
        ],
    )
    def gather_kernel(table_hbm, idx_hbm, out_hbm, idx_v, rows_v, sem):
        wid = lax.axis_index("s") * _NUM_CORES + lax.axis_index("c")
        base = wid * rows_per_worker
        for c in range(n_chunks):
            off = base + c * chunk
            pltpu.sync_copy(idx_hbm.at[pl.ds(off, chunk)], idx_v)
            pltpu.async_copy(table_hbm.at[idx_v], rows_v, sem).wait()
            pltpu.sync_copy(rows_v, out_hbm.at[pl.ds(off, chunk)])

    return gather_kernel


def kernel(words, chars, table):
    del chars
    batch, hist = words.shape
    total = batch * hist
    flat = words.reshape(total)
    out = _make_gather(total, 1600)(table, flat)
    return out.reshape(batch, hist, _EMBED_DIM)


# speedup vs baseline: 1.0034x; 1.0034x over previous
import functools

import jax
import jax.numpy as jnp
from jax import lax
from jax.experimental import pallas as pl
from jax.experimental.pallas import tpu as pltpu
from jax.experimental.pallas import tpu_sc as plsc

_EMBED_DIM = 32
_NUM_CORES = 2
_NUM_SUBCORES = 16
_NUM_WORKERS = _NUM_CORES * _NUM_SUBCORES


def _make_gather(total_rows: int, chunk: int, nbuf: int):
    assert total_rows % (_NUM_WORKERS * chunk) == 0
    rows_per_worker = total_rows // _NUM_WORKERS
    n_chunks = rows_per_worker // chunk
    mesh = plsc.VectorSubcoreMesh(core_axis_name="c", subcore_axis_name="s")

    @functools.partial(
        pl.kernel,
        mesh=mesh,
        compiler_params=pltpu.CompilerParams(use_tc_tiling_on_sc=False),
        out_type=jax.ShapeDtypeStruct((total_rows, _EMBED_DIM), jnp.float32),
        scratch_types=[
            pltpu.VMEM((rows_per_worker,), jnp.int32),
            pltpu.VMEM((nbuf, chunk, _EMBED_DIM), jnp.float32),
            pltpu.SemaphoreType.DMA((nbuf,)),
            pltpu.SemaphoreType.DMA((nbuf,)),
        ],
    )
    def gather_kernel(table_hbm, idx_hbm, out_hbm, idx_v, rows_v, gat_sem, out_sem):
        wid = lax.axis_index("s") * _NUM_CORES + lax.axis_index("c")
        base = wid * rows_per_worker
        pltpu.sync_copy(idx_hbm.at[pl.ds(base, rows_per_worker)], idx_v)

        def start_gather(c, b):
            return pltpu.async_copy(
                table_hbm.at[idx_v.at[pl.ds(c * chunk, chunk)]],
                rows_v.at[b], gat_sem.at[b])

        gat = {}
        out = {}
        for b in range(min(nbuf, n_chunks)):
            gat[b] = start_gather(b, b)
        for c in range(n_chunks):
            b = c % nbuf
            gat[b].wait()
            out[b] = pltpu.async_copy(
                rows_v.at[b], out_hbm.at[pl.ds(base + c * chunk, chunk)],
                out_sem.at[b])
            nxt = c + nbuf
            if nxt < n_chunks:
                out[b].wait()
                gat[b] = start_gather(nxt, b)
        for c in range(max(0, n_chunks - nbuf), n_chunks):
            b = c % nbuf
            if b in out:
                out[b].wait()
                del out[b]

    return gather_kernel


def kernel(words, chars, table):
    del chars
    batch, hist = words.shape
    total = batch * hist
    flat = words.reshape(total)
    out = _make_gather(total, 1280, 3)(table, flat)
    return out.reshape(batch, hist, _EMBED_DIM)
